# Initial kernel scaffold; baseline (speedup 1.0000x reference)
import jax
import jax.numpy as jnp
from jax.experimental import pallas as pl


def kernel(x, w1_aug, w2_aug):
    raise NotImplementedError("write your pallas kernel here")


# speedup vs baseline: 4.6553x; 4.6553x over previous
import jax
import jax.numpy as jnp
from jax.experimental import pallas as pl
from jax.experimental.pallas import tpu as pltpu

_D_IN = 12
_ACTIONS = 4
_TILE_B = 2048


def _mlp_kernel(x_ref, w1_ref, w2_ref, o_ref):
    w1 = w1_ref[...]
    h = jnp.dot(x_ref[...], w1[:_D_IN, :], preferred_element_type=jnp.float32)
    h = jnp.maximum(h + w1[_D_IN:_D_IN + 1, :], 0.0)
    o = jnp.dot(h, w2_ref[...], preferred_element_type=jnp.float32)
    o_ref[...] = o[:, :_ACTIONS]


def kernel(x, w1_aug, w2_aug):
    x = jnp.asarray(x, jnp.float32)
    B = x.shape[0]
    B_pad = ((B + _TILE_B - 1) // _TILE_B) * _TILE_B
    if B_pad != B:
        x = jnp.pad(x, ((0, B_pad - B), (0, 0)))
    out = pl.pallas_call(
        _mlp_kernel,
        out_shape=jax.ShapeDtypeStruct((B_pad, _ACTIONS), jnp.float32),
        grid=(B_pad // _TILE_B,),
        in_specs=[
            pl.BlockSpec((_TILE_B, _D_IN), lambda i: (i, 0)),
            pl.BlockSpec((16, 128), lambda i: (0, 0)),
            pl.BlockSpec((128, 128), lambda i: (0, 0)),
        ],
        out_specs=pl.BlockSpec((_TILE_B, _ACTIONS), lambda i: (i, 0)),
        compiler_params=pltpu.CompilerParams(
            dimension_semantics=("parallel",)
        ),
    )(x, w1_aug, w2_aug)
    return out[:B]
